# Initial kernel scaffold; baseline (speedup 1.0000x reference)
import jax
import jax.numpy as jnp
from jax.experimental import pallas as pl


def kernel(x, tables):
    raise NotImplementedError("write your pallas kernel here")


# speedup vs baseline: 1.2652x; 1.2652x over previous
import functools

import jax
import jax.numpy as jnp
from jax import lax
from jax.experimental import pallas as pl
from jax.experimental.pallas import tpu as pltpu
from jax.experimental.pallas import tpu_sc as plsc

N_FIELDS = 26
VOCAB = 100000
EMBED_DIM = 32
BATCH = 16384

_LANES = 16
_GROUP = 128


def _build_sc_kernel():
    info = plsc.get_sparse_core_info()
    nc, ns = info.num_cores, info.num_subcores
    nw = nc * ns
    rows_total = BATCH * N_FIELDS
    assert rows_total % nw == 0
    rows_per_w = rows_total // nw
    assert rows_per_w % _GROUP == 0
    groups_per_w = rows_per_w // _GROUP
    g_per_s = 8
    assert groups_per_w % g_per_s == 0
    n_super = groups_per_w // g_per_s
    rows_per_s = g_per_s * _GROUP
    assert rows_per_w % N_FIELDS == 0

    mesh = plsc.VectorSubcoreMesh(core_axis_name="c", subcore_axis_name="s")

    @functools.partial(
        pl.kernel,
        out_type=jax.ShapeDtypeStruct((rows_total, EMBED_DIM), jnp.float32),
        mesh=mesh,
        scratch_types=[
            pltpu.VMEM((groups_per_w, _GROUP), jnp.int32),
            pltpu.VMEM((g_per_s, _GROUP), jnp.int32),
            pltpu.VMEM((rows_per_s, EMBED_DIM), jnp.float32),
            pltpu.SemaphoreType.DMA,
        ],
        compiler_params=pltpu.CompilerParams(use_tc_tiling_on_sc=False),
    )
    def sc_kernel(tab_hbm, x_hbm, out_hbm, xv, gidx, rows, sem):
        wid = lax.axis_index("c") * ns + lax.axis_index("s")
        pltpu.sync_copy(x_hbm.at[wid], xv)
        iota = lax.iota(jnp.int32, 16)
        row_base = wid * rows_per_w

        def super_body(s, carry):
            for r in range(g_per_s):
                g = s * g_per_s + r
                for ch in range(_GROUP // _LANES):
                    j = (g * _GROUP + ch * _LANES) + iota
                    fld = lax.rem(j, N_FIELDS)
                    raw = xv[g, pl.ds(ch * _LANES, _LANES)]
                    gidx[r, pl.ds(ch * _LANES, _LANES)] = raw + fld * VOCAB
            copies = []
            for r in range(g_per_s):
                copies.append(
                    pltpu.async_copy(
                        tab_hbm.at[gidx.at[r]],
                        rows.at[pl.ds(r * _GROUP, _GROUP)],
                        sem,
                    )
                )
            for c in copies:
                c.wait()
            pltpu.sync_copy(
                rows, out_hbm.at[pl.ds(row_base + s * rows_per_s, rows_per_s)]
            )
            return carry

        lax.fori_loop(0, n_super, super_body, 0)

    return sc_kernel, nw, groups_per_w


def kernel(x, tables):
    sc_kernel, nw, groups_per_w = _build_sc_kernel()
    tab_flat = tables.reshape(N_FIELDS * VOCAB, EMBED_DIM)
    x_flat = x.reshape(nw, groups_per_w, _GROUP)
    out = sc_kernel(tab_flat, x_flat)
    return out.reshape(BATCH, N_FIELDS * EMBED_DIM)
